# baseline (speedup 1.0000x reference)
import math

import jax
import jax.numpy as jnp
from jax.experimental import pallas as pl

_D = 768
_SEGS = ((0, 64), (64, 96), (96, 112))
_S_PAD = 128
_TILE = 512


def _attn_kernel(q_ref, k_ref, v_ref, b_ref, o_ref):
    q = q_ref[...]
    k = k_ref[...]
    s = jax.lax.dot_general(
        q, k, (((1,), (1,)), ((), ())),
        preferred_element_type=jnp.float32,
        precision=jax.lax.Precision.HIGHEST,
    ) * (1.0 / math.sqrt(_D))
    s = s + b_ref[...]
    col = jax.lax.broadcasted_iota(jnp.int32, s.shape, 1)
    p = jnp.zeros_like(s)
    for lo, hi in _SEGS:
        m = (col >= lo) & (col < hi)
        sm = jnp.where(m, s, -1e30)
        mx = jnp.max(sm, axis=1, keepdims=True)
        e = jnp.exp(sm - mx)
        p = p + e / jnp.sum(e, axis=1, keepdims=True)
    p = p * (1.0 / 3.0)
    o_ref[...] = jax.lax.dot_general(
        p, v_ref[...], (((1,), (0,)), ((), ())),
        preferred_element_type=jnp.float32,
        precision=jax.lax.Precision.HIGHEST,
    )


def kernel(query, keys0, values0, salience0, keys1, values1, salience1,
           keys2, values2, salience2):
    B, T, D = query.shape
    n = B * T
    q = query.reshape(n, D)
    k = jnp.concatenate([keys0, keys1, keys2], axis=0)
    v = jnp.concatenate([values0, values1, values2], axis=0)
    pad = _S_PAD - k.shape[0]
    k = jnp.pad(k, ((0, pad), (0, 0)))
    v = jnp.pad(v, ((0, pad), (0, 0)))
    bias = jnp.pad(
        jnp.concatenate([salience0, salience1, salience2]), (0, pad)
    ).reshape(1, _S_PAD)

    out = pl.pallas_call(
        _attn_kernel,
        grid=(n // _TILE,),
        in_specs=[
            pl.BlockSpec((_TILE, D), lambda i: (i, 0)),
            pl.BlockSpec((_S_PAD, D), lambda i: (0, 0)),
            pl.BlockSpec((_S_PAD, D), lambda i: (0, 0)),
            pl.BlockSpec((1, _S_PAD), lambda i: (0, 0)),
        ],
        out_specs=pl.BlockSpec((_TILE, D), lambda i: (i, 0)),
        out_shape=jax.ShapeDtypeStruct((n, D), jnp.float32),
    )(q, k, v, bias)
    return out.reshape(B, T, D)


# speedup vs baseline: 2.7667x; 2.7667x over previous
import math

import jax
import jax.numpy as jnp
from jax.experimental import pallas as pl

_D = 768
_SEGS = ((0, 64), (64, 96), (96, 112))
_S_PAD = 128
_TILE = 512


def _attn_kernel(q_ref, k_ref, v_ref, b_ref, o_ref):
    q = q_ref[...].astype(jnp.bfloat16)
    k = k_ref[...]
    s = jax.lax.dot_general(
        q, k, (((1,), (1,)), ((), ())),
        preferred_element_type=jnp.float32,
    ) * (1.0 / math.sqrt(_D))
    s = s + b_ref[...]
    col = jax.lax.broadcasted_iota(jnp.int32, s.shape, 1)
    p = jnp.zeros_like(s)
    for lo, hi in _SEGS:
        m = (col >= lo) & (col < hi)
        sm = jnp.where(m, s, -1e30)
        mx = jnp.max(sm, axis=1, keepdims=True)
        e = jnp.exp(sm - mx)
        p = p + e / jnp.sum(e, axis=1, keepdims=True)
    p = (p * (1.0 / 3.0)).astype(jnp.bfloat16)
    o_ref[...] = jax.lax.dot_general(
        p, v_ref[...], (((1,), (0,)), ((), ())),
        preferred_element_type=jnp.float32,
    )


def kernel(query, keys0, values0, salience0, keys1, values1, salience1,
           keys2, values2, salience2):
    B, T, D = query.shape
    n = B * T
    q = query.reshape(n, D)
    k = jnp.concatenate([keys0, keys1, keys2], axis=0)
    v = jnp.concatenate([values0, values1, values2], axis=0)
    pad = _S_PAD - k.shape[0]
    k = jnp.pad(k, ((0, pad), (0, 0))).astype(jnp.bfloat16)
    v = jnp.pad(v, ((0, pad), (0, 0))).astype(jnp.bfloat16)
    bias = jnp.pad(
        jnp.concatenate([salience0, salience1, salience2]), (0, pad)
    ).reshape(1, _S_PAD)

    out = pl.pallas_call(
        _attn_kernel,
        grid=(n // _TILE,),
        in_specs=[
            pl.BlockSpec((_TILE, D), lambda i: (i, 0)),
            pl.BlockSpec((_S_PAD, D), lambda i: (0, 0)),
            pl.BlockSpec((_S_PAD, D), lambda i: (0, 0)),
            pl.BlockSpec((1, _S_PAD), lambda i: (0, 0)),
        ],
        out_specs=pl.BlockSpec((_TILE, D), lambda i: (i, 0)),
        out_shape=jax.ShapeDtypeStruct((n, D), jnp.float32),
    )(q, k, v, bias)
    return out.reshape(B, T, D)


# speedup vs baseline: 3.0471x; 1.1013x over previous
import math

import jax
import jax.numpy as jnp
from jax.experimental import pallas as pl

_D = 768
_SEGS = ((0, 64), (64, 96), (96, 112))
_S_PAD = 128
_TILE = 512


def _attn_kernel(q_ref, k_ref, v_ref, b_ref, o_ref):
    q = q_ref[...].astype(jnp.bfloat16)
    k = k_ref[...]
    s = jax.lax.dot_general(
        q, k, (((1,), (1,)), ((), ())),
        preferred_element_type=jnp.float32,
    )
    s = s + b_ref[...]
    mx = jnp.max(s, axis=1, keepdims=True)
    e = jnp.exp(s - mx)
    col = jax.lax.broadcasted_iota(jnp.int32, s.shape, 1)
    denom = jnp.zeros_like(s)
    for lo, hi in _SEGS:
        m = (col >= lo) & (col < hi)
        seg_sum = jnp.sum(jnp.where(m, e, 0.0), axis=1, keepdims=True)
        denom = denom + jnp.where(m, seg_sum, 0.0)
    p = (e / jnp.maximum(denom, 1e-30)).astype(jnp.bfloat16)
    o_ref[...] = jax.lax.dot_general(
        p, v_ref[...], (((1,), (0,)), ((), ())),
        preferred_element_type=jnp.float32,
    )


def kernel(query, keys0, values0, salience0, keys1, values1, salience1,
           keys2, values2, salience2):
    B, T, D = query.shape
    n = B * T
    q = query.reshape(n, D)
    k = jnp.concatenate([keys0, keys1, keys2], axis=0)
    v = jnp.concatenate([values0, values1, values2], axis=0)
    pad = _S_PAD - k.shape[0]
    k = (jnp.pad(k, ((0, pad), (0, 0))) * (1.0 / math.sqrt(_D))).astype(jnp.bfloat16)
    v = (jnp.pad(v, ((0, pad), (0, 0))) * (1.0 / 3.0)).astype(jnp.bfloat16)
    bias = jnp.pad(
        jnp.concatenate([salience0, salience1, salience2]), (0, pad),
        constant_values=-1e30,
    ).reshape(1, _S_PAD)

    out = pl.pallas_call(
        _attn_kernel,
        grid=(n // _TILE,),
        in_specs=[
            pl.BlockSpec((_TILE, D), lambda i: (i, 0)),
            pl.BlockSpec((_S_PAD, D), lambda i: (0, 0)),
            pl.BlockSpec((_S_PAD, D), lambda i: (0, 0)),
            pl.BlockSpec((1, _S_PAD), lambda i: (0, 0)),
        ],
        out_specs=pl.BlockSpec((_TILE, D), lambda i: (i, 0)),
        out_shape=jax.ShapeDtypeStruct((n, D), jnp.float32),
    )(q, k, v, bias)
    return out.reshape(B, T, D)


# speedup vs baseline: 3.7398x; 1.2273x over previous
import math

import jax
import jax.numpy as jnp
from jax.experimental import pallas as pl

_D = 768
_SEGS = ((0, 64), (64, 96), (96, 112))
_S_PAD = 128
_TILE = 1024


def _attn_kernel(q_ref, k_ref, v_ref, b_ref, o_ref):
    q = q_ref[...].astype(jnp.bfloat16)
    k = k_ref[...]
    s = jax.lax.dot_general(
        q, k, (((1,), (1,)), ((), ())),
        preferred_element_type=jnp.float32,
    )
    s = s + b_ref[...]
    mx = jnp.max(s, axis=1, keepdims=True)
    e = jnp.exp(s - mx)
    col = jax.lax.broadcasted_iota(jnp.int32, s.shape, 1)
    denom = jnp.zeros_like(s)
    for lo, hi in _SEGS:
        m = (col >= lo) & (col < hi)
        seg_sum = jnp.sum(jnp.where(m, e, 0.0), axis=1, keepdims=True)
        denom = denom + jnp.where(m, seg_sum, 0.0)
    p = (e / jnp.maximum(denom, 1e-30)).astype(jnp.bfloat16)
    o_ref[...] = jax.lax.dot_general(
        p, v_ref[...], (((1,), (0,)), ((), ())),
        preferred_element_type=jnp.float32,
    )


def kernel(query, keys0, values0, salience0, keys1, values1, salience1,
           keys2, values2, salience2):
    B, T, D = query.shape
    n = B * T
    q = query.reshape(n, D)
    k = jnp.concatenate([keys0, keys1, keys2], axis=0)
    v = jnp.concatenate([values0, values1, values2], axis=0)
    pad = _S_PAD - k.shape[0]
    k = (jnp.pad(k, ((0, pad), (0, 0))) * (1.0 / math.sqrt(_D))).astype(jnp.bfloat16)
    v = (jnp.pad(v, ((0, pad), (0, 0))) * (1.0 / 3.0)).astype(jnp.bfloat16)
    bias = jnp.pad(
        jnp.concatenate([salience0, salience1, salience2]), (0, pad),
        constant_values=-1e30,
    ).reshape(1, _S_PAD)

    out = pl.pallas_call(
        _attn_kernel,
        grid=(n // _TILE,),
        in_specs=[
            pl.BlockSpec((_TILE, D), lambda i: (i, 0)),
            pl.BlockSpec((_S_PAD, D), lambda i: (0, 0)),
            pl.BlockSpec((_S_PAD, D), lambda i: (0, 0)),
            pl.BlockSpec((1, _S_PAD), lambda i: (0, 0)),
        ],
        out_specs=pl.BlockSpec((_TILE, D), lambda i: (i, 0)),
        out_shape=jax.ShapeDtypeStruct((n, D), jnp.float32),
    )(q, k, v, bias)
    return out.reshape(B, T, D)


# speedup vs baseline: 4.1491x; 1.1094x over previous
import math

import jax
import jax.numpy as jnp
from jax.experimental import pallas as pl

_D = 768
_SEGS = ((0, 64), (64, 96), (96, 112))
_S_PAD = 128
_TILE = 2048


def _attn_kernel(q_ref, k_ref, v_ref, b_ref, o_ref):
    q = q_ref[...].astype(jnp.bfloat16)
    k = k_ref[...]
    s = jax.lax.dot_general(
        q, k, (((1,), (1,)), ((), ())),
        preferred_element_type=jnp.float32,
    )
    s = s + b_ref[...]
    mx = jnp.max(s, axis=1, keepdims=True)
    e = jnp.exp(s - mx)
    col = jax.lax.broadcasted_iota(jnp.int32, s.shape, 1)
    denom = jnp.zeros_like(s)
    for lo, hi in _SEGS:
        m = (col >= lo) & (col < hi)
        seg_sum = jnp.sum(jnp.where(m, e, 0.0), axis=1, keepdims=True)
        denom = denom + jnp.where(m, seg_sum, 0.0)
    p = (e / jnp.maximum(denom, 1e-30)).astype(jnp.bfloat16)
    o_ref[...] = jax.lax.dot_general(
        p, v_ref[...], (((1,), (0,)), ((), ())),
        preferred_element_type=jnp.float32,
    )


def kernel(query, keys0, values0, salience0, keys1, values1, salience1,
           keys2, values2, salience2):
    B, T, D = query.shape
    n = B * T
    q = query.reshape(n, D)
    k = jnp.concatenate([keys0, keys1, keys2], axis=0)
    v = jnp.concatenate([values0, values1, values2], axis=0)
    pad = _S_PAD - k.shape[0]
    k = (jnp.pad(k, ((0, pad), (0, 0))) * (1.0 / math.sqrt(_D))).astype(jnp.bfloat16)
    v = (jnp.pad(v, ((0, pad), (0, 0))) * (1.0 / 3.0)).astype(jnp.bfloat16)
    bias = jnp.pad(
        jnp.concatenate([salience0, salience1, salience2]), (0, pad),
        constant_values=-1e30,
    ).reshape(1, _S_PAD)

    out = pl.pallas_call(
        _attn_kernel,
        grid=(n // _TILE,),
        in_specs=[
            pl.BlockSpec((_TILE, D), lambda i: (i, 0)),
            pl.BlockSpec((_S_PAD, D), lambda i: (0, 0)),
            pl.BlockSpec((_S_PAD, D), lambda i: (0, 0)),
            pl.BlockSpec((1, _S_PAD), lambda i: (0, 0)),
        ],
        out_specs=pl.BlockSpec((_TILE, D), lambda i: (i, 0)),
        out_shape=jax.ShapeDtypeStruct((n, D), jnp.float32),
    )(q, k, v, bias)
    return out.reshape(B, T, D)
